# baseline (speedup 1.0000x reference)
import jax
import jax.numpy as jnp
from jax.experimental import pallas as pl

_SCALING = 16.0 / 8.0
_OUT = 2048
_PACK = 2176


def _fused_kernel(x_ref, wc_ref, b_ref, bb_ref, o_ref):
    xt = x_ref[...].astype(jnp.bfloat16)
    big = jax.lax.dot_general(
        xt, wc_ref[...], (((1,), (1,)), ((), ())),
        preferred_element_type=jnp.float32)
    base = big[:, :_OUT]
    xa = big[:, _OUT:_OUT + 8]
    l0 = big[:, _OUT + 8:_OUT + 9]
    l1 = big[:, _OUT + 9:_OUT + 10]
    w = jnp.where(l0 >= l1, jax.nn.sigmoid(l0 - l1), 0.0) * _SCALING
    xa = (xa * w).astype(jnp.bfloat16)
    lora = jax.lax.dot_general(
        xa, bb_ref[...], (((1,), (1,)), ((), ())),
        preferred_element_type=jnp.float32)
    o_ref[...] = base + b_ref[...] + lora


def kernel(x, base_W, base_b, gate_W, lora_A_W, lora_B_W):
    n_tokens, in_f = x.shape
    out_f = base_W.shape[0]
    tm = 512
    grid = (n_tokens // tm,)
    w_cat = jnp.concatenate(
        [base_W, lora_A_W, gate_W,
         jnp.zeros((_PACK - out_f - 10, in_f), jnp.float32)],
        axis=0).astype(jnp.bfloat16)
    bias2d = base_b.reshape(1, out_f)
    bb16 = lora_B_W.astype(jnp.bfloat16)
    return pl.pallas_call(
        _fused_kernel,
        grid=grid,
        in_specs=[
            pl.BlockSpec((tm, in_f), lambda i: (i, 0)),
            pl.BlockSpec((_PACK, in_f), lambda i: (0, 0)),
            pl.BlockSpec((1, out_f), lambda i: (0, 0)),
            pl.BlockSpec((out_f, 8), lambda i: (0, 0)),
        ],
        out_specs=pl.BlockSpec((tm, out_f), lambda i: (i, 0)),
        out_shape=jax.ShapeDtypeStruct((n_tokens, out_f), jnp.float32),
    )(x, w_cat, bias2d, bb16)


# speedup vs baseline: 1.4059x; 1.4059x over previous
import jax
import jax.numpy as jnp
from jax.experimental import pallas as pl
from jax.experimental.pallas import tpu as pltpu

_SCALING = 16.0 / 8.0


def _fused_kernel(x_ref, w_ref, b_ref, g_ref, a_ref, bb_ref, o_ref):
    xt = x_ref[...]
    base = jax.lax.dot_general(
        xt, w_ref[...], (((1,), (1,)), ((), ())),
        preferred_element_type=jnp.float32)
    logits = jax.lax.dot_general(
        xt, g_ref[...], (((1,), (1,)), ((), ())),
        preferred_element_type=jnp.float32)
    l0 = logits[:, 0:1]
    l1 = logits[:, 1:2]
    w = jnp.where(l0 >= l1, jax.nn.sigmoid(l0 - l1), 0.0) * _SCALING
    xa = jax.lax.dot_general(
        xt, a_ref[...], (((1,), (1,)), ((), ())),
        preferred_element_type=jnp.float32)
    xa = xa * w
    lora = jax.lax.dot_general(
        xa, bb_ref[...], (((1,), (1,)), ((), ())),
        preferred_element_type=jnp.float32)
    o_ref[...] = base + b_ref[...] + lora


def kernel(x, base_W, base_b, gate_W, lora_A_W, lora_B_W):
    n_tokens, in_f = x.shape
    out_f = base_W.shape[0]
    tm = 512
    grid = (n_tokens // tm,)
    bias2d = base_b.reshape(1, out_f)
    return pl.pallas_call(
        _fused_kernel,
        grid=grid,
        in_specs=[
            pl.BlockSpec((tm, in_f), lambda i: (i, 0)),
            pl.BlockSpec((out_f, in_f), lambda i: (0, 0)),
            pl.BlockSpec((1, out_f), lambda i: (0, 0)),
            pl.BlockSpec(gate_W.shape, lambda i: (0, 0)),
            pl.BlockSpec(lora_A_W.shape, lambda i: (0, 0)),
            pl.BlockSpec(lora_B_W.shape, lambda i: (0, 0)),
        ],
        out_specs=pl.BlockSpec((tm, out_f), lambda i: (i, 0)),
        out_shape=jax.ShapeDtypeStruct((n_tokens, out_f), jnp.float32),
        compiler_params=pltpu.CompilerParams(
            dimension_semantics=(pltpu.PARALLEL,)),
    )(x, base_W, bias2d, gate_W, lora_A_W, lora_B_W)


# speedup vs baseline: 1.4268x; 1.0148x over previous
import jax
import jax.numpy as jnp
from jax.experimental import pallas as pl
from jax.experimental.pallas import tpu as pltpu

_SCALING = 16.0 / 8.0


def _fused_kernel(x_ref, w_ref, b_ref, g_ref, a_ref, bb_ref, o_ref):
    xt = x_ref[...]
    base = jax.lax.dot_general(
        xt, w_ref[...], (((1,), (1,)), ((), ())),
        preferred_element_type=jnp.float32)
    logits = jax.lax.dot_general(
        xt, g_ref[...], (((1,), (1,)), ((), ())),
        preferred_element_type=jnp.float32)
    l0 = logits[:, 0:1]
    l1 = logits[:, 1:2]
    w = jnp.where(l0 >= l1, jax.nn.sigmoid(l0 - l1), 0.0) * _SCALING
    xa = jax.lax.dot_general(
        xt, a_ref[...], (((1,), (1,)), ((), ())),
        preferred_element_type=jnp.float32)
    xa = xa * w
    lora = jax.lax.dot_general(
        xa, bb_ref[...], (((1,), (1,)), ((), ())),
        preferred_element_type=jnp.float32)
    o_ref[...] = base + b_ref[...] + lora


def kernel(x, base_W, base_b, gate_W, lora_A_W, lora_B_W):
    n_tokens, in_f = x.shape
    out_f = base_W.shape[0]
    tm = 1024
    grid = (n_tokens // tm,)
    bias2d = base_b.reshape(1, out_f)
    return pl.pallas_call(
        _fused_kernel,
        grid=grid,
        in_specs=[
            pl.BlockSpec((tm, in_f), lambda i: (i, 0)),
            pl.BlockSpec((out_f, in_f), lambda i: (0, 0)),
            pl.BlockSpec((1, out_f), lambda i: (0, 0)),
            pl.BlockSpec(gate_W.shape, lambda i: (0, 0)),
            pl.BlockSpec(lora_A_W.shape, lambda i: (0, 0)),
            pl.BlockSpec(lora_B_W.shape, lambda i: (0, 0)),
        ],
        out_specs=pl.BlockSpec((tm, out_f), lambda i: (i, 0)),
        out_shape=jax.ShapeDtypeStruct((n_tokens, out_f), jnp.float32),
        compiler_params=pltpu.CompilerParams(
            dimension_semantics=(pltpu.PARALLEL,)),
    )(x, base_W, bias2d, gate_W, lora_A_W, lora_B_W)
